# Initial kernel scaffold; baseline (speedup 1.0000x reference)
import jax
import jax.numpy as jnp
from jax.experimental import pallas as pl


def kernel(x, Wf, bf, Wb, bb, Wr, br, Wo, bo, Wx, bx, Wfc, bfc, Wc, bc, Wd, bd, Wp, bp):
    raise NotImplementedError("write your pallas kernel here")


# speedup vs baseline: 7.1958x; 7.1958x over previous
import functools

import jax
import jax.numpy as jnp
from jax.experimental import pallas as pl

_B, _CIN, _H, _W = 2, 20, 256, 256
_A = 9
_PRE_TOPK, _POST_TOPK = 2000, 200
_RPN_NMS, _DET_NMS, _SCORE_TH = 0.5, 0.3, 0.15


def _conv2d(x, w, b, pad):
    y = jax.lax.conv_general_dilated(
        x, w, (1, 1), ((pad, pad), (pad, pad)),
        dimension_numbers=('NCHW', 'OIHW', 'NCHW'))
    return y + b[None, :, None, None]


def _make_anchors(h, w):
    scales = jnp.array([8.0, 16.0, 32.0], dtype=jnp.float32)
    ratios = jnp.array([0.5, 1.0, 2.0], dtype=jnp.float32)
    ws = (scales[:, None] / jnp.sqrt(ratios[None, :])).reshape(-1)
    hs = (scales[:, None] * jnp.sqrt(ratios[None, :])).reshape(-1)
    cy, cx = jnp.meshgrid(jnp.arange(h, dtype=jnp.float32) + 0.5,
                          jnp.arange(w, dtype=jnp.float32) + 0.5, indexing='ij')
    cx = cx[..., None]
    cy = cy[..., None]
    x1 = cx - ws / 2
    y1 = cy - hs / 2
    x2 = cx + ws / 2
    y2 = cy + hs / 2
    return jnp.stack([x1, y1, x2, y2], axis=-1).reshape(-1, 4)


def _decode(anchors, deltas, hh, ww):
    aw = anchors[:, 2] - anchors[:, 0]
    ah = anchors[:, 3] - anchors[:, 1]
    acx = anchors[:, 0] + 0.5 * aw
    acy = anchors[:, 1] + 0.5 * ah
    dx, dy, dw, dh = deltas[:, 0], deltas[:, 1], deltas[:, 2], deltas[:, 3]
    px = acx + dx * aw
    py = acy + dy * ah
    pw = aw * jnp.exp(dw)
    ph = ah * jnp.exp(dh)
    x1 = jnp.clip(px - 0.5 * pw, 0.0, ww - 1.0)
    x2 = jnp.clip(px + 0.5 * pw, 0.0, ww - 1.0)
    y1 = jnp.clip(py - 0.5 * ph, 0.0, hh - 1.0)
    y2 = jnp.clip(py + 0.5 * ph, 0.0, hh - 1.0)
    return jnp.stack([x1, y1, x2, y2], axis=1)


def _nms_body(n_valid, thr, coords_ref, keep_ref):
    x1 = coords_ref[0]
    y1 = coords_ref[1]
    x2 = coords_ref[2]
    y2 = coords_ref[3]
    rows = x1.shape[0]
    area = (x2 - x1) * (y2 - y1)
    jr = (jax.lax.broadcasted_iota(jnp.int32, (rows, 128), 0) * 128
          + jax.lax.broadcasted_iota(jnp.int32, (rows, 128), 1)).astype(jnp.float32)

    def body(i, supp):
        fi = i.astype(jnp.float32)
        oh = (jr == fi).astype(jnp.float32)
        bx1 = jnp.sum(x1 * oh)
        by1 = jnp.sum(y1 * oh)
        bx2 = jnp.sum(x2 * oh)
        by2 = jnp.sum(y2 * oh)
        s_i = jnp.sum(supp * oh)
        barea = (bx2 - bx1) * (by2 - by1)
        iw = jnp.maximum(jnp.minimum(x2, bx2) - jnp.maximum(x1, bx1), 0.0)
        ih = jnp.maximum(jnp.minimum(y2, by2) - jnp.maximum(y1, by1), 0.0)
        inter = iw * ih
        iou = inter / (area + barea - inter + 1e-9)
        newly = jnp.where((iou > thr) & (jr > fi), 1.0, 0.0)
        return jnp.where(s_i < 0.5, jnp.maximum(supp, newly), supp)

    supp = jax.lax.fori_loop(0, n_valid, body,
                             jnp.zeros((rows, 128), jnp.float32))
    keep_ref[...] = 1.0 - supp


def _nms_pallas(boxes, n, thr):
    np_ = ((n + 127) // 128) * 128
    rows = np_ // 128
    b = jnp.pad(boxes, ((0, np_ - n), (0, 0)))
    coords = b.T.reshape(4, rows, 128)
    keep = pl.pallas_call(
        functools.partial(_nms_body, n, thr),
        out_shape=jax.ShapeDtypeStruct((rows, 128), jnp.float32),
    )(coords)
    return keep.reshape(-1)[:n] > 0.5


def _roi_align(feat, boxes, out=7):
    c, hh, ww = feat.shape
    r = boxes.shape[0]
    x1, y1, x2, y2 = boxes[:, 0], boxes[:, 1], boxes[:, 2], boxes[:, 3]
    gx = x1[:, None] + (jnp.arange(out, dtype=jnp.float32) + 0.5) / out * jnp.maximum(x2 - x1, 1.0)[:, None]
    gy = y1[:, None] + (jnp.arange(out, dtype=jnp.float32) + 0.5) / out * jnp.maximum(y2 - y1, 1.0)[:, None]
    X = jnp.broadcast_to(gx[:, None, :], (r, out, out))
    Y = jnp.broadcast_to(gy[:, :, None], (r, out, out))
    x0 = jnp.clip(jnp.floor(X), 0, ww - 2).astype(jnp.int32)
    y0 = jnp.clip(jnp.floor(Y), 0, hh - 2).astype(jnp.int32)
    wx = jnp.clip(X - x0.astype(jnp.float32), 0.0, 1.0)
    wy = jnp.clip(Y - y0.astype(jnp.float32), 0.0, 1.0)
    v00 = feat[:, y0, x0]
    v01 = feat[:, y0, x0 + 1]
    v10 = feat[:, y0 + 1, x0]
    v11 = feat[:, y0 + 1, x0 + 1]
    val = v00 * (1 - wy) * (1 - wx) + v01 * (1 - wy) * wx + v10 * wy * (1 - wx) + v11 * wy * wx
    return jnp.transpose(val, (1, 0, 2, 3))


def kernel(x, Wf, bf, Wb, bb, Wr, br, Wo, bo, Wx, bx, Wfc, bfc, Wc, bc, Wd, bd, Wp, bp):
    bn = x.shape[0]
    feat = jax.nn.relu(_conv2d(x, Wf, bf, 1))
    feat = jax.nn.relu(_conv2d(feat, Wb, bb, 1))
    anchors = _make_anchors(_H, _W)
    rpn_in = jax.nn.relu(_conv2d(feat, Wr, br, 1))
    obj = _conv2d(rpn_in, Wo, bo, 0)
    bbx = _conv2d(rpn_in, Wx, bx, 0)
    obj = jnp.transpose(obj, (0, 2, 3, 1)).reshape(bn, -1)
    bbx = jnp.transpose(bbx, (0, 2, 3, 1)).reshape(bn, -1, 4)
    outs = []
    for i in range(bn):
        logit, idx = jax.lax.top_k(obj[i], _PRE_TOPK)
        sco = jax.nn.sigmoid(logit)
        dec = _decode(anchors[idx], bbx[i][idx], _H, _W)
        keep = _nms_pallas(dec, _PRE_TOPK, _RPN_NMS)
        masked = jnp.where(keep, sco, -jnp.inf)
        _, idx2 = jax.lax.top_k(masked, _POST_TOPK)
        props = dec[idx2]
        roi = _roi_align(feat[i], props, 7)
        hdn = jax.nn.relu(roi.reshape(_POST_TOPK, -1) @ Wfc.T + bfc)
        cls = hdn @ Wc.T + bc
        deltas = hdn @ Wd.T + bd
        pbr = hdn @ Wp.T + bp
        boxes2 = _decode(props, deltas, _H, _W) + pbr
        score = jax.nn.softmax(cls, axis=-1)[:, 1]
        order = jnp.argsort(-score)
        bs = boxes2[order]
        ss = score[order]
        keep2 = _nms_pallas(bs, _POST_TOPK, _DET_NMS)
        fs = jnp.where(jnp.logical_and(keep2, ss > _SCORE_TH), ss, 0.0)
        outs.append(jnp.concatenate([bs, fs[:, None]], axis=1))
    return jnp.stack(outs, axis=0)
